# Initial kernel scaffold; baseline (speedup 1.0000x reference)
import jax
import jax.numpy as jnp
from jax.experimental import pallas as pl


def kernel(inputs, edge_index, weight):
    raise NotImplementedError("write your pallas kernel here")


# speedup vs baseline: 4.3645x; 4.3645x over previous
import functools

import jax
import jax.numpy as jnp
from jax import lax
from jax.experimental import pallas as pl
from jax.experimental.pallas import tpu as pltpu
from jax.experimental.pallas import tpu_sc as plsc

T, N, D = 4, 10000, 128
E = 320000

NUM_SC = 2
NUM_TILES = 16
T_PER_SC = T // NUM_SC
E_PER_TILE = E // NUM_TILES
CHUNK = 80
N_CHUNKS = E_PER_TILE // CHUNK
WB_ROWS = 80
ROWS_MAJOR = 640


def _mm_body(x_ref, w_ref, o_ref):
    o_ref[...] = jnp.dot(x_ref[...], w_ref[...],
                         preferred_element_type=jnp.float32)


def _project(inputs_flat, weight):
    bn = 2000
    grid = (inputs_flat.shape[0] // bn,)
    return pl.pallas_call(
        _mm_body,
        grid=grid,
        in_specs=[
            pl.BlockSpec((bn, inputs_flat.shape[1]), lambda i: (i, 0)),
            pl.BlockSpec(weight.shape, lambda i: (0, 0)),
        ],
        out_specs=pl.BlockSpec((bn, D), lambda i: (i, 0)),
        out_shape=jax.ShapeDtypeStruct((inputs_flat.shape[0], D), jnp.float32),
    )(inputs_flat, weight)


def _sc_body(support_hbm, rows_hbm, cols_hbm, out_hbm,
             rows_v, cols_v, gath_v, wb_v, zb_v, acc_sh, sem):
    c = lax.axis_index("c")
    s = lax.axis_index("s")

    zeros16 = jnp.zeros((16,), jnp.float32)

    def _zb_zero(r, _):
        for j in range(D // 16):
            zb_v[r, pl.ds(j * 16, 16)] = zeros16
        return 0
    lax.fori_loop(0, WB_ROWS, _zb_zero, 0)

    wb_base = s * ROWS_MAJOR
    n_wb = jnp.where(s < NUM_TILES - 1,
                     ROWS_MAJOR // WB_ROWS,
                     (N - (NUM_TILES - 1) * ROWS_MAJOR) // WB_ROWS)

    for ti in range(T_PER_SC):
        t = c * T_PER_SC + ti
        coff = (t * N).astype(jnp.int32)
        ebase = t * E + s * E_PER_TILE

        def _zero(p, _):
            r0 = wb_base + p * WB_ROWS
            pltpu.sync_copy(zb_v, acc_sh.at[pl.ds(r0, WB_ROWS)])
            return 0
        lax.fori_loop(0, n_wb, _zero, 0)
        plsc.subcore_barrier()

        def _chunk(i, _):
            off = ebase + i * CHUNK
            pltpu.sync_copy(rows_hbm.at[pl.ds(off, CHUNK)], rows_v)
            pltpu.sync_copy(cols_hbm.at[pl.ds(off, CHUNK)], cols_v)
            for j in range(CHUNK // 16):
                sl = pl.ds(j * 16, 16)
                cols_v[sl] = cols_v[sl] + coff
            pltpu.async_copy(support_hbm.at[cols_v], gath_v, sem).wait()
            pltpu.sync_copy(gath_v, acc_sh.at[rows_v], add=True)
            return 0
        lax.fori_loop(0, N_CHUNKS, _chunk, 0)
        plsc.subcore_barrier()

        def _wb(p, _):
            r0 = wb_base + p * WB_ROWS
            pltpu.sync_copy(acc_sh.at[pl.ds(r0, WB_ROWS)], wb_v)

            def _relu(r, _):
                for j in range(D // 16):
                    sl = pl.ds(j * 16, 16)
                    wb_v[r, sl] = jnp.maximum(wb_v[r, sl], 0.0)
                return 0
            lax.fori_loop(0, WB_ROWS, _relu, 0)
            pltpu.sync_copy(wb_v, out_hbm.at[t, pl.ds(r0, WB_ROWS), :])
            return 0
        lax.fori_loop(0, n_wb, _wb, 0)


def _aggregate(support_flat, rows, cols):
    mesh = plsc.VectorSubcoreMesh(core_axis_name="c", subcore_axis_name="s")
    f = functools.partial(
        pl.kernel,
        out_type=jax.ShapeDtypeStruct((T, N, D), jnp.float32),
        mesh=mesh,
        scratch_types=[
            pltpu.VMEM((CHUNK,), jnp.int32),
            pltpu.VMEM((CHUNK,), jnp.int32),
            pltpu.VMEM((CHUNK, D), jnp.float32),
            pltpu.VMEM((WB_ROWS, D), jnp.float32),
            pltpu.VMEM((WB_ROWS, D), jnp.float32),
            pltpu.VMEM_SHARED((N, D), jnp.float32),
            pltpu.SemaphoreType.DMA,
        ],
    )(_sc_body)
    return f(support_flat, rows, cols)


def kernel(inputs, edge_index, weight):
    inputs_flat = inputs.reshape(T * N, inputs.shape[-1])
    support_flat = _project(inputs_flat, weight)
    rows = edge_index[:, 0, :].astype(jnp.int32).reshape(T * E)
    cols = edge_index[:, 1, :].astype(jnp.int32).reshape(T * E)
    return _aggregate(support_flat, rows, cols)


# speedup vs baseline: 10.6409x; 2.4381x over previous
import functools

import jax
import jax.numpy as jnp
from jax import lax
from jax.experimental import pallas as pl
from jax.experimental.pallas import tpu as pltpu
from jax.experimental.pallas import tpu_sc as plsc

T, N, D = 4, 10000, 128
E = 320000

NUM_SC = 2
NUM_TILES = 16
T_PER_SC = T // NUM_SC
E_PER_TILE = E // NUM_TILES
CHUNK = 80
BLK_EDGES = 4000
N_BLKS = E_PER_TILE // BLK_EDGES
BLK_CHUNKS = BLK_EDGES // CHUNK
WB_ROWS = 80
ROWS_MAJOR = 640


def _mm_body(x_ref, w_ref, o_ref):
    o_ref[...] = jnp.dot(x_ref[...], w_ref[...],
                         preferred_element_type=jnp.float32)


def _project(inputs_flat, weight):
    bn = 2000
    grid = (inputs_flat.shape[0] // bn,)
    return pl.pallas_call(
        _mm_body,
        grid=grid,
        in_specs=[
            pl.BlockSpec((bn, inputs_flat.shape[1]), lambda i: (i, 0)),
            pl.BlockSpec(weight.shape, lambda i: (0, 0)),
        ],
        out_specs=pl.BlockSpec((bn, D), lambda i: (i, 0)),
        out_shape=jax.ShapeDtypeStruct((inputs_flat.shape[0], D), jnp.float32),
    )(inputs_flat, weight)


def _sc_body(support_hbm, rows_hbm, cols_hbm, out_hbm,
             rows_st, cols_st, rows_v0, rows_v1, gath_v0, gath_v1,
             wb_v, acc_sh, semg0, semg1):
    c = lax.axis_index("c")
    s = lax.axis_index("s")

    zeros16 = jnp.zeros((16,), jnp.float32)

    wb_base = s * ROWS_MAJOR
    n_wb = jnp.where(s < NUM_TILES - 1,
                     ROWS_MAJOR // WB_ROWS,
                     (N - (NUM_TILES - 1) * ROWS_MAJOR) // WB_ROWS)

    def _col_slice(i):
        off = pl.multiple_of(i * CHUNK, 8)
        return cols_st.at[pl.ds(off, CHUNK)]

    def _gather(i, gath_v, sem):
        return pltpu.async_copy(support_hbm.at[_col_slice(i)], gath_v, sem)

    def _gather_wait(i, gath_v, sem):
        pltpu.make_async_copy(support_hbm.at[_col_slice(i)], gath_v, sem).wait()

    def _prep_rows(rows_v, i):
        for j in range(CHUNK // 16):
            off = pl.multiple_of(i * CHUNK + j * 16, 8)
            rows_v[pl.ds(j * 16, 16)] = rows_st[pl.ds(off, 16)]

    for ti in range(T_PER_SC):
        t = c * T_PER_SC + ti
        ebase = t * E + s * E_PER_TILE

        def _wb_zero(r, _):
            for j in range(D // 16):
                wb_v[r, pl.ds(j * 16, 16)] = zeros16
            return 0
        lax.fori_loop(0, WB_ROWS, _wb_zero, 0)

        def _zero(p, _):
            r0 = wb_base + p * WB_ROWS
            pltpu.sync_copy(wb_v, acc_sh.at[pl.ds(r0, WB_ROWS)])
            return 0
        lax.fori_loop(0, n_wb, _zero, 0)
        plsc.subcore_barrier()

        for b in range(N_BLKS):
            bbase = ebase + b * BLK_EDGES
            pltpu.sync_copy(rows_hbm.at[pl.ds(bbase, BLK_EDGES)], rows_st)
            pltpu.sync_copy(cols_hbm.at[pl.ds(bbase, BLK_EDGES)], cols_st)

            _gather(0, gath_v0, semg0)

            def _pipe(k, _):
                i0 = 2 * k
                i1 = 2 * k + 1
                _gather(i1, gath_v1, semg1)
                _prep_rows(rows_v0, i0)
                _gather_wait(i0, gath_v0, semg0)
                pltpu.sync_copy(gath_v0, acc_sh.at[rows_v0], add=True)

                @pl.when(i1 + 1 < BLK_CHUNKS)
                def _():
                    _gather(i1 + 1, gath_v0, semg0)
                _prep_rows(rows_v1, i1)
                _gather_wait(i1, gath_v1, semg1)
                pltpu.sync_copy(gath_v1, acc_sh.at[rows_v1], add=True)
                return 0
            lax.fori_loop(0, BLK_CHUNKS // 2, _pipe, 0)
        plsc.subcore_barrier()

        def _wb(p, _):
            r0 = wb_base + p * WB_ROWS
            pltpu.sync_copy(acc_sh.at[pl.ds(r0, WB_ROWS)], wb_v)

            def _relu(r, _):
                for j in range(D // 16):
                    sl = pl.ds(j * 16, 16)
                    wb_v[r, sl] = jnp.maximum(wb_v[r, sl], 0.0)
                return 0
            lax.fori_loop(0, WB_ROWS, _relu, 0)
            pltpu.sync_copy(wb_v, out_hbm.at[t, pl.ds(r0, WB_ROWS), :])
            return 0
        lax.fori_loop(0, n_wb, _wb, 0)


def _aggregate(support_flat, rows, cols):
    mesh = plsc.VectorSubcoreMesh(core_axis_name="c", subcore_axis_name="s")
    f = functools.partial(
        pl.kernel,
        out_type=jax.ShapeDtypeStruct((T, N, D), jnp.float32),
        mesh=mesh,
        scratch_types=[
            pltpu.VMEM((BLK_EDGES,), jnp.int32),
            pltpu.VMEM((BLK_EDGES,), jnp.int32),
            pltpu.VMEM((CHUNK,), jnp.int32),
            pltpu.VMEM((CHUNK,), jnp.int32),
            pltpu.VMEM((CHUNK, D), jnp.float32),
            pltpu.VMEM((CHUNK, D), jnp.float32),
            pltpu.VMEM((WB_ROWS, D), jnp.float32),
            pltpu.VMEM_SHARED((N, D), jnp.float32),
            pltpu.SemaphoreType.DMA,
            pltpu.SemaphoreType.DMA,
        ],
    )(_sc_body)
    return f(support_flat, rows, cols)


def kernel(inputs, edge_index, weight):
    inputs_flat = inputs.reshape(T * N, inputs.shape[-1])
    support_flat = _project(inputs_flat, weight)
    rows = edge_index[:, 0, :].astype(jnp.int32).reshape(T * E)
    toff = (jnp.arange(T, dtype=jnp.int64) * N)[:, None]
    cols = (edge_index[:, 1, :] + toff).astype(jnp.int32).reshape(T * E)
    return _aggregate(support_flat, rows, cols)
